# baseline (speedup 1.0000x reference)
import functools

import jax
import jax.numpy as jnp
from jax import lax
from jax.experimental import pallas as pl
from jax.experimental.pallas import tpu as pltpu
from jax.experimental.pallas import tpu_sc as plsc

VOCAB = 1000000
DIM = 64
BATCH = 16384
QROWS = VOCAB // 2
NC = 2
NS = 16
L = 16
NW = NC * NS
BPW = BATCH // NW
C = 128
NCH = BPW // C
GPC = C // L

VB = 1024
K0_GRID = (VOCAB + VB - 1) // VB


def _pack_kernel(lo_ref, hi_ref, out_ref):
    tl = jnp.transpose(lo_ref[...], (1, 0))
    th = jnp.transpose(hi_ref[...], (1, 0))
    out_ref[...] = jnp.concatenate([tl, th], axis=1)


_pack = pl.pallas_call(
    _pack_kernel,
    grid=(K0_GRID,),
    in_specs=[
        pl.BlockSpec((DIM, VB // 2), lambda i: (0, 2 * i)),
        pl.BlockSpec((DIM, VB // 2), lambda i: (0, 2 * i + 1)),
    ],
    out_specs=pl.BlockSpec((VB // 2, 2 * DIM), lambda i: (i, 0)),
    out_shape=jax.ShapeDtypeStruct((K0_GRID * VB // 2, 2 * DIM), jnp.float32),
)

_mesh = plsc.VectorSubcoreMesh(
    core_axis_name="c", subcore_axis_name="s", num_cores=NC, num_subcores=NS
)


@functools.partial(
    pl.kernel,
    out_type=jax.ShapeDtypeStruct((NW, 2, L), jnp.float32),
    mesh=_mesh,
    scratch_types=[
        pltpu.VMEM((NCH, C), jnp.int32),
        pltpu.VMEM((NCH, C), jnp.int32),
        pltpu.VMEM((NCH, C), jnp.int32),
        pltpu.VMEM((NCH, C), jnp.int32),
        pltpu.VMEM((NCH, C), jnp.int32),
        pltpu.VMEM((NCH, C), jnp.int32),
        pltpu.VMEM((NCH, C), jnp.int32),
        pltpu.VMEM((NCH, C), jnp.int32),
        pltpu.VMEM((C, 2 * DIM), jnp.float32),
        pltpu.VMEM((C, 2 * DIM), jnp.float32),
        pltpu.VMEM((C, 2 * DIM), jnp.float32),
        pltpu.VMEM((C, 2 * DIM), jnp.float32),
        pltpu.VMEM((2, L), jnp.float32),
        pltpu.SemaphoreType.DMA,
    ],
    compiler_params=pltpu.CompilerParams(
        use_tc_tiling_on_sc=True, needs_layout_passes=False
    ),
)
def _w2v_partials(qi_hbm, qo_hbm, qin_hbm, qon_hbm, li_hbm, lo_hbm, lin_hbm,
                  lon_hbm, xi_hbm, xo_hbm, out_hbm,
                  qi, qo, qin, qon, li, lo, lin, lon, bi, bo, bin_, bon,
                  ob, sem):
    wid = lax.axis_index("s") * NC + lax.axis_index("c")

    pltpu.sync_copy(qi_hbm.at[wid], qi)
    pltpu.sync_copy(qo_hbm.at[wid], qo)
    pltpu.sync_copy(qin_hbm.at[wid], qin)
    pltpu.sync_copy(qon_hbm.at[wid], qon)
    pltpu.sync_copy(li_hbm.at[wid], li)
    pltpu.sync_copy(lo_hbm.at[wid], lo)
    pltpu.sync_copy(lin_hbm.at[wid], lin)
    pltpu.sync_copy(lon_hbm.at[wid], lon)

    zeros = jnp.zeros((L,), jnp.float32)
    iota = lax.iota(jnp.int32, L)

    streams = ((xi_hbm, qi, li, bi), (xo_hbm, qo, lo, bo),
               (xi_hbm, qin, lin, bin_), (xo_hbm, qon, lon, bon))

    acc1 = zeros
    acc2 = zeros
    for j in range(NCH):
        cps = [pltpu.async_copy(x.at[q.at[j]], buf, sem)
               for x, q, _, buf in streams]
        for cp in cps:
            cp.wait()

        def group_body(g, accs):
            a1, a2 = accs
            rows = g * L + iota
            offs = [lref[j, pl.ds(g * L, L)] for _, _, lref, _ in streams]

            def d_body(d, accs2):
                b1, b2 = accs2
                gi = plsc.load_gather(bi, [rows, offs[0] + d])
                go = plsc.load_gather(bo, [rows, offs[1] + d])
                gn = plsc.load_gather(bin_, [rows, offs[2] + d])
                gq = plsc.load_gather(bon, [rows, offs[3] + d])
                return (b1 + gi * go, b2 + gn * gq)

            return lax.fori_loop(0, DIM, d_body, (a1, a2))

        acc1, acc2 = lax.fori_loop(0, GPC, group_body, (acc1, acc2))

    ob[0, :] = acc1
    ob[1, :] = acc2
    pltpu.sync_copy(ob, out_hbm.at[wid])


def kernel(iword, oword, inword, onword, ivectors_table, ovectors_table):
    ti = ivectors_table.T
    to = ovectors_table.T
    xi = _pack(ti, ti)
    xo = _pack(to, to)

    def _q(v):
        return ((v >> 10) << 9) + (v & 511)

    def _l(v):
        return ((v >> 9) & 1) << 6

    qi = _q(iword).reshape(NW, NCH, C)
    qo = _q(oword).reshape(NW, NCH, C)
    qin = _q(inword).reshape(NW, NCH, C)
    qon = _q(onword).reshape(NW, NCH, C)
    li = _l(iword).reshape(NW, NCH, C)
    lo = _l(oword).reshape(NW, NCH, C)
    lin = _l(inword).reshape(NW, NCH, C)
    lon = _l(onword).reshape(NW, NCH, C)
    parts = _w2v_partials(qi, qo, qin, qon, li, lo, lin, lon, xi, xo)
    s1 = parts[:, 0, :].sum()
    s2 = parts[:, 1, :].sum()
    oloss = jnp.log(jnp.clip(jax.nn.sigmoid(s1), 1e-12, 1.0))
    nloss = jnp.log(jnp.clip(jax.nn.sigmoid(-s2), 1e-12, 1.0))
    return -(oloss + nloss)


# speedup vs baseline: 1.0822x; 1.0822x over previous
import functools

import jax
import jax.numpy as jnp
from jax import lax
from jax.experimental import pallas as pl
from jax.experimental.pallas import tpu as pltpu
from jax.experimental.pallas import tpu_sc as plsc

VOCAB = 1000000
DIM = 64
BATCH = 16384
QROWS = VOCAB // 2
NC = 2
NS = 16
L = 16
NW = NC * NS
BPW = BATCH // NW
C = 128
NCH = BPW // C
GPC = C // L

VB = 1024
K0_GRID = (VOCAB + VB - 1) // VB


def _pack_kernel(lo_ref, hi_ref, out_ref):
    stacked = jnp.concatenate([lo_ref[...], hi_ref[...]], axis=0)
    out_ref[...] = jnp.transpose(stacked, (1, 0))


_pack = pl.pallas_call(
    _pack_kernel,
    grid=(K0_GRID,),
    in_specs=[
        pl.BlockSpec((DIM, VB // 2), lambda i: (0, 2 * i)),
        pl.BlockSpec((DIM, VB // 2), lambda i: (0, 2 * i + 1)),
    ],
    out_specs=pl.BlockSpec((VB // 2, 2 * DIM), lambda i: (i, 0)),
    out_shape=jax.ShapeDtypeStruct((K0_GRID * VB // 2, 2 * DIM), jnp.float32),
)

_mesh = plsc.VectorSubcoreMesh(
    core_axis_name="c", subcore_axis_name="s", num_cores=NC, num_subcores=NS
)


@functools.partial(
    pl.kernel,
    out_type=jax.ShapeDtypeStruct((NW, 2, L), jnp.float32),
    mesh=_mesh,
    scratch_types=[
        pltpu.VMEM((NCH, C), jnp.int32),
        pltpu.VMEM((NCH, C), jnp.int32),
        pltpu.VMEM((NCH, C), jnp.int32),
        pltpu.VMEM((NCH, C), jnp.int32),
        pltpu.VMEM((NCH, C), jnp.int32),
        pltpu.VMEM((NCH, C), jnp.int32),
        pltpu.VMEM((NCH, C), jnp.int32),
        pltpu.VMEM((NCH, C), jnp.int32),
        pltpu.VMEM((C, 2 * DIM), jnp.float32),
        pltpu.VMEM((C, 2 * DIM), jnp.float32),
        pltpu.VMEM((C, 2 * DIM), jnp.float32),
        pltpu.VMEM((C, 2 * DIM), jnp.float32),
        pltpu.VMEM((2, L), jnp.float32),
        pltpu.SemaphoreType.DMA,
    ],
    compiler_params=pltpu.CompilerParams(
        use_tc_tiling_on_sc=True, needs_layout_passes=False
    ),
)
def _w2v_partials(qi_hbm, qo_hbm, qin_hbm, qon_hbm, li_hbm, lo_hbm, lin_hbm,
                  lon_hbm, xi_hbm, xo_hbm, out_hbm,
                  qi, qo, qin, qon, li, lo, lin, lon, bi, bo, bin_, bon,
                  ob, sem):
    wid = lax.axis_index("s") * NC + lax.axis_index("c")

    pltpu.sync_copy(qi_hbm.at[wid], qi)
    pltpu.sync_copy(qo_hbm.at[wid], qo)
    pltpu.sync_copy(qin_hbm.at[wid], qin)
    pltpu.sync_copy(qon_hbm.at[wid], qon)
    pltpu.sync_copy(li_hbm.at[wid], li)
    pltpu.sync_copy(lo_hbm.at[wid], lo)
    pltpu.sync_copy(lin_hbm.at[wid], lin)
    pltpu.sync_copy(lon_hbm.at[wid], lon)

    zeros = jnp.zeros((L,), jnp.float32)
    iota = lax.iota(jnp.int32, L)

    streams = ((xi_hbm, qi, li, bi), (xo_hbm, qo, lo, bo),
               (xi_hbm, qin, lin, bin_), (xo_hbm, qon, lon, bon))

    acc1 = zeros
    acc2 = zeros
    for j in range(NCH):
        cps = [pltpu.async_copy(x.at[q.at[j]], buf, sem)
               for x, q, _, buf in streams]
        for cp in cps:
            cp.wait()

        def group_body(g, accs):
            a1, a2 = accs
            rows = g * L + iota
            offs = [lref[j, pl.ds(g * L, L)] for _, _, lref, _ in streams]

            def d_body(d, accs2):
                b1, b2 = accs2
                gi = plsc.load_gather(bi, [rows, offs[0] + d])
                go = plsc.load_gather(bo, [rows, offs[1] + d])
                gn = plsc.load_gather(bin_, [rows, offs[2] + d])
                gq = plsc.load_gather(bon, [rows, offs[3] + d])
                return (b1 + gi * go, b2 + gn * gq)

            return lax.fori_loop(0, DIM, d_body, (a1, a2))

        acc1, acc2 = lax.fori_loop(0, GPC, group_body, (acc1, acc2))

    ob[0, :] = acc1
    ob[1, :] = acc2
    pltpu.sync_copy(ob, out_hbm.at[wid])


def kernel(iword, oword, inword, onword, ivectors_table, ovectors_table):
    ti = ivectors_table.T
    to = ovectors_table.T
    xi = _pack(ti, ti)
    xo = _pack(to, to)

    hb = VB // 2

    def _q(v):
        return (v // VB) * hb + (v % hb)

    def _l(v):
        return ((v % VB) // hb) << 6

    qi = _q(iword).reshape(NW, NCH, C)
    qo = _q(oword).reshape(NW, NCH, C)
    qin = _q(inword).reshape(NW, NCH, C)
    qon = _q(onword).reshape(NW, NCH, C)
    li = _l(iword).reshape(NW, NCH, C)
    lo = _l(oword).reshape(NW, NCH, C)
    lin = _l(inword).reshape(NW, NCH, C)
    lon = _l(onword).reshape(NW, NCH, C)
    parts = _w2v_partials(qi, qo, qin, qon, li, lo, lin, lon, xi, xo)
    s1 = parts[:, 0, :].sum()
    s2 = parts[:, 1, :].sum()
    oloss = jnp.log(jnp.clip(jax.nn.sigmoid(s1), 1e-12, 1.0))
    nloss = jnp.log(jnp.clip(jax.nn.sigmoid(-s2), 1e-12, 1.0))
    return -(oloss + nloss)


# speedup vs baseline: 1.3324x; 1.2312x over previous
import functools

import jax
import jax.numpy as jnp
from jax import lax
from jax.experimental import pallas as pl
from jax.experimental.pallas import tpu as pltpu
from jax.experimental.pallas import tpu_sc as plsc

DIM = 64
BATCH = 16384
NC = 2
NS = 16
L = 16
NW = NC * NS
BPW = BATCH // NW
C = 128
NCH = BPW // C
VPR = DIM // L

_mesh = plsc.VectorSubcoreMesh(
    core_axis_name="c", subcore_axis_name="s", num_cores=NC, num_subcores=NS
)


@functools.partial(
    pl.kernel,
    out_type=jax.ShapeDtypeStruct((NW, 2, L), jnp.float32),
    mesh=_mesh,
    scratch_types=[
        pltpu.VMEM((NCH, C), jnp.int32),
        pltpu.VMEM((NCH, C), jnp.int32),
        pltpu.VMEM((NCH, C), jnp.int32),
        pltpu.VMEM((NCH, C), jnp.int32),
        pltpu.VMEM((C, DIM), jnp.float32),
        pltpu.VMEM((C, DIM), jnp.float32),
        pltpu.VMEM((C, DIM), jnp.float32),
        pltpu.VMEM((C, DIM), jnp.float32),
        pltpu.VMEM((2, L), jnp.float32),
        pltpu.SemaphoreType.DMA,
    ],
    compiler_params=pltpu.CompilerParams(
        use_tc_tiling_on_sc=False, skip_device_barrier=True
    ),
)
def _w2v_partials(iw_hbm, ow_hbm, inw_hbm, onw_hbm, itab_hbm, otab_hbm,
                  out_hbm, idx_i, idx_o, idx_in, idx_on, ri, ro, rin, ron,
                  ob, sem):
    wid = lax.axis_index("s") * NC + lax.axis_index("c")

    pltpu.sync_copy(iw_hbm.at[wid], idx_i)
    pltpu.sync_copy(ow_hbm.at[wid], idx_o)
    pltpu.sync_copy(inw_hbm.at[wid], idx_in)
    pltpu.sync_copy(onw_hbm.at[wid], idx_on)

    zeros = jnp.zeros((L,), jnp.float32)
    acc1 = [zeros] * VPR
    acc2 = [zeros] * VPR

    for j in range(NCH):
        cps = [
            pltpu.async_copy(itab_hbm.at[idx_i.at[j]], ri, sem),
            pltpu.async_copy(otab_hbm.at[idx_o.at[j]], ro, sem),
            pltpu.async_copy(itab_hbm.at[idx_in.at[j]], rin, sem),
            pltpu.async_copy(otab_hbm.at[idx_on.at[j]], ron, sem),
        ]
        for cp in cps:
            cp.wait()

        def row_body(r, accs):
            a1, a2 = accs
            a1 = tuple(
                a1[p] + ri[r, pl.ds(p * L, L)] * ro[r, pl.ds(p * L, L)]
                for p in range(VPR)
            )
            a2 = tuple(
                a2[p] + rin[r, pl.ds(p * L, L)] * ron[r, pl.ds(p * L, L)]
                for p in range(VPR)
            )
            return (a1, a2)

        acc1, acc2 = lax.fori_loop(0, C, row_body, (tuple(acc1), tuple(acc2)))

    t1 = (acc1[0] + acc1[1]) + (acc1[2] + acc1[3])
    t2 = (acc2[0] + acc2[1]) + (acc2[2] + acc2[3])
    ob[0, :] = t1
    ob[1, :] = t2
    pltpu.sync_copy(ob, out_hbm.at[wid])


def kernel(iword, oword, inword, onword, ivectors_table, ovectors_table):
    iw = iword.reshape(NW, NCH, C)
    ow = oword.reshape(NW, NCH, C)
    inw = inword.reshape(NW, NCH, C)
    onw = onword.reshape(NW, NCH, C)
    parts = _w2v_partials(iw, ow, inw, onw, ivectors_table, ovectors_table)
    s1 = parts[:, 0, :].sum()
    s2 = parts[:, 1, :].sum()
    oloss = jnp.log(jnp.clip(jax.nn.sigmoid(s1), 1e-12, 1.0))
    nloss = jnp.log(jnp.clip(jax.nn.sigmoid(-s2), 1e-12, 1.0))
    return -(oloss + nloss)
